# baseline (speedup 1.0000x reference)
import jax
import jax.numpy as jnp
from jax import lax
from jax.experimental import pallas as pl
from jax.experimental.pallas import tpu as pltpu
from jax.experimental.pallas import tpu_sc as plsc

_G = 8
_S = 16
_L = 16


def _oracle_body(losses_hbm, gidx_hbm, out_hbm, idx_v, vals_v, out_v, sem):
    c = lax.axis_index("c")
    s = lax.axis_index("s")

    @pl.when(jnp.logical_and(c == 0, s == 0))
    def _():
        pltpu.sync_copy(gidx_hbm, idx_v)
        pltpu.async_copy(losses_hbm.at[idx_v], vals_v, sem).wait()

        m = jnp.float32(-jnp.inf)
        for g in range(_G):
            v = vals_v[pl.ds(g * _S, _S)]
            m = jnp.maximum(m, jnp.sum(v))
        m = m * jnp.float32(1.0 / _S)
        out_v[...] = jnp.full((_L,), m, jnp.float32)
        pltpu.sync_copy(out_v, out_hbm)


@jax.jit
def _oracle_max(losses, gidx):
    mesh = plsc.VectorSubcoreMesh(
        core_axis_name="c", subcore_axis_name="s", num_cores=2, num_subcores=16
    )
    run = pl.kernel(
        _oracle_body,
        out_type=jax.ShapeDtypeStruct((_L,), jnp.float32),
        mesh=mesh,
        scratch_types=[
            pltpu.VMEM((_G * _S,), jnp.int32),
            pltpu.VMEM((_G * _S,), jnp.float32),
            pltpu.VMEM((_L,), jnp.float32),
            pltpu.SemaphoreType.DMA,
        ],
        compiler_params=pltpu.CompilerParams(needs_layout_passes=False),
    )
    return run(losses, gidx)[0]


def kernel(losses, groups):
    gidx = groups.reshape(-1).astype(jnp.int32)
    return _oracle_max(losses, gidx)


# speedup vs baseline: 1.0720x; 1.0720x over previous
import jax
import jax.numpy as jnp
from jax import lax
from jax.experimental import pallas as pl
from jax.experimental.pallas import tpu as pltpu
from jax.experimental.pallas import tpu_sc as plsc

_G = 8
_S = 16
_L = 16


def _oracle_body(losses_hbm, gidx_hbm, out_hbm, idx_v, vals_v, out_v, sem):
    c = lax.axis_index("c")
    s = lax.axis_index("s")

    @pl.when(jnp.logical_and(c == 0, s == 0))
    def _():
        pltpu.sync_copy(gidx_hbm, idx_v)
        pltpu.async_copy(losses_hbm.at[idx_v], vals_v, sem).wait()

        m = jnp.float32(-jnp.inf)
        for g in range(_G):
            v = vals_v[pl.ds(g * _S, _S)]
            m = jnp.maximum(m, jnp.sum(v))
        m = m * jnp.float32(1.0 / _S)
        out_v[...] = jnp.full((_L,), m, jnp.float32)
        pltpu.sync_copy(out_v, out_hbm)


@jax.jit
def _oracle_max(losses, gidx):
    mesh = plsc.VectorSubcoreMesh(
        core_axis_name="c", subcore_axis_name="s", num_cores=1, num_subcores=16
    )
    run = pl.kernel(
        _oracle_body,
        out_type=jax.ShapeDtypeStruct((_L,), jnp.float32),
        mesh=mesh,
        scratch_types=[
            pltpu.VMEM((_G * _S,), jnp.int32),
            pltpu.VMEM((_G * _S,), jnp.float32),
            pltpu.VMEM((_L,), jnp.float32),
            pltpu.SemaphoreType.DMA,
        ],
        compiler_params=pltpu.CompilerParams(
            needs_layout_passes=False,
            disable_bounds_checks=True,
            disable_semaphore_checks=True,
            skip_device_barrier=True,
        ),
    )
    return run(losses, gidx)[0]


def kernel(losses, groups):
    gidx = groups.reshape(-1).astype(jnp.int32)
    return _oracle_max(losses, gidx)
